# baseline (speedup 1.0000x reference)
import dataclasses

import jax
import jax.numpy as jnp
from jax import lax
from jax.experimental import pallas as pl
from jax.experimental.pallas import tpu as pltpu
from jax.experimental.pallas import tpu_sc as plsc

_GRID = 1024
_NY = _GRID * _GRID
_NQ = 2097152
_NCORES = 2
_NSUB = 16
_NW = _NCORES * _NSUB
_QPW = _NQ // _NW
_C = 2048
_NCH = _QPW // _C
_W = 128
_R = _C // _W
_L = 16
_VPR = _W // _L

_BCH = 4096
_BHALO = 8200
_BIN = _BCH + _BHALO
_BOFFMAX = _NY - _BIN
_SLAB = _NY // _NW


def _worker_id():
    return lax.axis_index("s") * _NCORES + lax.axis_index("c")


def _build_table_kernel(y_hbm, q_hbm, yb, qb, sem):
    del sem
    base = _worker_id() * _SLAB

    @pl.loop(0, _SLAB // _BCH)
    def _chunk(c):
        cb = base + c * _BCH
        off = jnp.minimum(cb, _BOFFMAX)
        sh = cb - off
        pltpu.sync_copy(y_hbm.at[pl.ds(off, _BIN)], yb)

        @pl.loop(0, _BCH // _L)
        def _pack(k):
            lanes = lax.iota(jnp.int32, _L)
            m = cb + k * _L + lanes
            lloc = sh + k * _L + lanes
            v0 = yb[pl.ds(sh + k * _L, _L)]
            dj = jnp.where((m & 127) == 127, 897, 1)
            di = jnp.where(((m >> 7) & 7) == 7, 7296, 128)
            s1 = jnp.minimum(lloc + dj, _BIN - 1)
            s2 = jnp.minimum(lloc + di, _BIN - 1)
            s3 = jnp.minimum(lloc + di + dj, _BIN - 1)
            v1 = plsc.load_gather(yb, [s1])
            v2 = plsc.load_gather(yb, [s2])
            v3 = plsc.load_gather(yb, [s3])
            q = k * _L + lanes
            plsc.store_scatter(qb, [q, jnp.full((_L,), 0, jnp.int32)], v0)
            plsc.store_scatter(qb, [q, jnp.full((_L,), 1, jnp.int32)], v1)
            plsc.store_scatter(qb, [q, jnp.full((_L,), 2, jnp.int32)], v2)
            plsc.store_scatter(qb, [q, jnp.full((_L,), 3, jnp.int32)], v3)

        pltpu.sync_copy(qb, q_hbm.at[pl.ds(cb, _BCH)])


def _interp_kernel_flat(q_hbm, x_hbm, o_hbm, xb, i00, i01, i10, i11,
                        v00, v01, v10, v11, t0b, t1b, ob, sem):
    base = _worker_id() * _QPW

    @pl.loop(0, _NCH)
    def _chunk(ch):
        qbase = base + ch * _C
        pltpu.sync_copy(x_hbm.at[pl.ds(2 * qbase, 2 * _C)], xb)

        @pl.loop(0, _C // _L)
        def _build(k):
            row = k // _VPR
            col = (k % _VPR) * _L
            xoff = 256 * (k // 8) + _L * (k % 8)
            x0 = xb[pl.ds(xoff, _L)]
            x1 = xb[pl.ds(xoff + 128, _L)]
            u = jnp.clip(x0, 0.0, 1.0) * float(_GRID - 1)
            v = jnp.clip(x1, 0.0, 1.0) * float(_GRID - 1)
            iv = jnp.minimum(u.astype(jnp.int32), _GRID - 2)
            jv = jnp.minimum(v.astype(jnp.int32), _GRID - 2)
            sl = pl.ds(k * _L, _L)
            t0b[sl] = u - iv.astype(jnp.float32)
            t1b[sl] = v - jv.astype(jnp.float32)
            p00 = ((iv >> 3) << 13) + ((jv >> 7) << 10) + ((iv & 7) << 7) \
                + (jv & 127)
            q4 = 4 * p00
            csl = pl.ds(col, _L)
            i00[row, csl] = q4
            i01[row, csl] = q4 + 1
            i10[row, csl] = q4 + 2
            i11[row, csl] = q4 + 3

        copies = []
        for r in range(_R):
            copies.append(pltpu.async_copy(q_hbm.at[i00.at[r]], v00.at[r], sem))
            copies.append(pltpu.async_copy(q_hbm.at[i01.at[r]], v01.at[r], sem))
            copies.append(pltpu.async_copy(q_hbm.at[i10.at[r]], v10.at[r], sem))
            copies.append(pltpu.async_copy(q_hbm.at[i11.at[r]], v11.at[r], sem))
        for cp_ in copies:
            cp_.wait()

        @pl.loop(0, _C // _L)
        def _interp(k):
            row = k // _VPR
            csl = pl.ds((k % _VPR) * _L, _L)
            a = v00[row, csl]
            b = v01[row, csl]
            c = v10[row, csl]
            d = v11[row, csl]
            sl = pl.ds(k * _L, _L)
            tv = t1b[sl]
            top = a + tv * (b - a)
            bot = c + tv * (d - c)
            ob[sl] = top + t0b[sl] * (bot - top)

        pltpu.sync_copy(ob, o_hbm.at[pl.ds(qbase, _C)])


def _interp_kernel(q_hbm, x_hbm, o_hbm, xb, i00, v4, t0b, t1b, ob, sem):
    base = _worker_id() * _QPW

    @pl.loop(0, _NCH)
    def _chunk(ch):
        qbase = base + ch * _C
        pltpu.sync_copy(x_hbm.at[pl.ds(2 * qbase, 2 * _C)], xb)

        @pl.loop(0, _C // _L)
        def _build(k):
            row = k // _VPR
            col = (k % _VPR) * _L
            xoff = 256 * (k // 8) + _L * (k % 8)
            x0 = xb[pl.ds(xoff, _L)]
            x1 = xb[pl.ds(xoff + 128, _L)]
            u = jnp.clip(x0, 0.0, 1.0) * float(_GRID - 1)
            v = jnp.clip(x1, 0.0, 1.0) * float(_GRID - 1)
            iv = jnp.minimum(u.astype(jnp.int32), _GRID - 2)
            jv = jnp.minimum(v.astype(jnp.int32), _GRID - 2)
            sl = pl.ds(k * _L, _L)
            t0b[sl] = u - iv.astype(jnp.float32)
            t1b[sl] = v - jv.astype(jnp.float32)
            p00 = ((iv >> 3) << 13) + ((jv >> 7) << 10) + ((iv & 7) << 7) \
                + (jv & 127)
            i00[row, pl.ds(col, _L)] = p00

        copies = [
            pltpu.async_copy(q_hbm.at[i00.at[r]], v4.at[r], sem)
            for r in range(_R)
        ]
        for cp in copies:
            cp.wait()

        @pl.loop(0, _C // _L)
        def _interp(k):
            r = k // _VPR
            qlanes = (k % _VPR) * _L + lax.iota(jnp.int32, _L)
            rsp = jnp.full((_L,), r, jnp.int32)
            a = plsc.load_gather(v4, [rsp, qlanes, jnp.full((_L,), 0, jnp.int32)])
            b = plsc.load_gather(v4, [rsp, qlanes, jnp.full((_L,), 1, jnp.int32)])
            c = plsc.load_gather(v4, [rsp, qlanes, jnp.full((_L,), 2, jnp.int32)])
            d = plsc.load_gather(v4, [rsp, qlanes, jnp.full((_L,), 3, jnp.int32)])
            sl = pl.ds(k * _L, _L)
            tv = t1b[sl]
            top = a + tv * (b - a)
            bot = c + tv * (d - c)
            ob[sl] = top + t0b[sl] * (bot - top)

        pltpu.sync_copy(ob, o_hbm.at[pl.ds(qbase, _C)])


def kernel(y, xs0, xs1, x):
    del xs0, xs1
    y_flat = y.reshape(128, 8, 8, 128).transpose(0, 2, 1, 3).reshape(-1)
    x_flat = x.reshape(16384, 128, 2).transpose(0, 2, 1).reshape(-1)
    mesh = plsc.VectorSubcoreMesh(core_axis_name="c", subcore_axis_name="s")
    cp = pltpu.CompilerParams(use_tc_tiling_on_sc=False)
    if "needs_layout_passes" in pltpu.CompilerParams.__dataclass_fields__:
        cp = dataclasses.replace(cp, needs_layout_passes=False)

    build = pl.kernel(
        _build_table_kernel,
        out_type=jax.ShapeDtypeStruct((_NY, 4), jnp.float32),
        mesh=mesh,
        scratch_types=[
            pltpu.VMEM((_BIN,), jnp.float32),
            pltpu.VMEM((_BCH, 4), jnp.float32),
            pltpu.SemaphoreType.DMA,
        ],
        compiler_params=cp,
    )
    q_table = build(y_flat)

    interp = pl.kernel(
        _interp_kernel_flat,
        out_type=jax.ShapeDtypeStruct((_NQ,), jnp.float32),
        mesh=mesh,
        scratch_types=[
            pltpu.VMEM((2 * _C,), jnp.float32),
            pltpu.VMEM((_R, _W), jnp.int32),
            pltpu.VMEM((_R, _W), jnp.int32),
            pltpu.VMEM((_R, _W), jnp.int32),
            pltpu.VMEM((_R, _W), jnp.int32),
            pltpu.VMEM((_R, _W), jnp.float32),
            pltpu.VMEM((_R, _W), jnp.float32),
            pltpu.VMEM((_R, _W), jnp.float32),
            pltpu.VMEM((_R, _W), jnp.float32),
            pltpu.VMEM((_C,), jnp.float32),
            pltpu.VMEM((_C,), jnp.float32),
            pltpu.VMEM((_C,), jnp.float32),
            pltpu.SemaphoreType.DMA,
        ],
        compiler_params=cp,
    )
    return interp(q_table.reshape(-1), x_flat)


# speedup vs baseline: 2.1755x; 2.1755x over previous
import dataclasses

import jax
import jax.numpy as jnp
from jax import lax
from jax.experimental import pallas as pl
from jax.experimental.pallas import tpu as pltpu
from jax.experimental.pallas import tpu_sc as plsc

_GRID = 1024
_NY = _GRID * _GRID
_NQ = 2097152
_NCORES = 2
_NSUB = 16
_NW = _NCORES * _NSUB
_QPW = _NQ // _NW
_C = 2048
_NCH = _QPW // _C
_W = 128
_R = _C // _W
_L = 16
_VPR = _W // _L

_BCH = 4096
_BHALO = 8200
_BIN = _BCH + _BHALO
_BOFFMAX = _NY - _BIN
_SLAB = _NY // _NW


def _worker_id():
    return lax.axis_index("s") * _NCORES + lax.axis_index("c")


def _build_table_kernel(y_hbm, q_hbm, yb, qb, sem):
    del sem
    base = _worker_id() * _SLAB

    @pl.loop(0, _SLAB // _BCH)
    def _chunk(c):
        cb = base + c * _BCH
        off = jnp.minimum(cb, _BOFFMAX)
        sh = cb - off
        pltpu.sync_copy(y_hbm.at[pl.ds(off, _BIN)], yb)

        @pl.loop(0, _BCH // _L)
        def _pack(k):
            lanes = lax.iota(jnp.int32, _L)
            m = cb + k * _L + lanes
            lloc = sh + k * _L + lanes
            v0 = yb[pl.ds(sh + k * _L, _L)]
            dj = jnp.where((m & 127) == 127, 897, 1)
            di = jnp.where(((m >> 7) & 7) == 7, 7296, 128)
            s1 = jnp.minimum(lloc + dj, _BIN - 1)
            s2 = jnp.minimum(lloc + di, _BIN - 1)
            s3 = jnp.minimum(lloc + di + dj, _BIN - 1)
            v1 = plsc.load_gather(yb, [s1])
            v2 = plsc.load_gather(yb, [s2])
            v3 = plsc.load_gather(yb, [s3])
            q4 = 4 * (k * _L + lanes)
            plsc.store_scatter(qb, [q4], v0)
            plsc.store_scatter(qb, [q4 + 1], v1)
            plsc.store_scatter(qb, [q4 + 2], v2)
            plsc.store_scatter(qb, [q4 + 3], v3)

        pltpu.sync_copy(qb, q_hbm.at[pl.ds(4 * cb, 4 * _BCH)])


def _interp_kernel_flat(q_hbm, x_hbm, o_hbm, xb, i00, i01, i10, i11,
                        v00, v01, v10, v11, t0b, t1b, ob, sem):
    base = _worker_id() * _QPW

    @pl.loop(0, _NCH)
    def _chunk(ch):
        qbase = base + ch * _C
        pltpu.sync_copy(x_hbm.at[pl.ds(2 * qbase, 2 * _C)], xb)

        @pl.loop(0, _C // _L)
        def _build(k):
            row = k // _VPR
            col = (k % _VPR) * _L
            xoff = 256 * (k // 8) + _L * (k % 8)
            x0 = xb[pl.ds(xoff, _L)]
            x1 = xb[pl.ds(xoff + 128, _L)]
            u = jnp.clip(x0, 0.0, 1.0) * float(_GRID - 1)
            v = jnp.clip(x1, 0.0, 1.0) * float(_GRID - 1)
            iv = jnp.minimum(u.astype(jnp.int32), _GRID - 2)
            jv = jnp.minimum(v.astype(jnp.int32), _GRID - 2)
            sl = pl.ds(k * _L, _L)
            t0b[sl] = u - iv.astype(jnp.float32)
            t1b[sl] = v - jv.astype(jnp.float32)
            p00 = ((iv >> 3) << 13) + ((jv >> 7) << 10) + ((iv & 7) << 7) \
                + (jv & 127)
            q4 = 4 * p00
            csl = pl.ds(col, _L)
            i00[row, csl] = q4
            i01[row, csl] = q4 + 1
            i10[row, csl] = q4 + 2
            i11[row, csl] = q4 + 3

        copies = []
        for r in range(_R):
            copies.append(pltpu.async_copy(q_hbm.at[i00.at[r]], v00.at[r], sem))
            copies.append(pltpu.async_copy(q_hbm.at[i01.at[r]], v01.at[r], sem))
            copies.append(pltpu.async_copy(q_hbm.at[i10.at[r]], v10.at[r], sem))
            copies.append(pltpu.async_copy(q_hbm.at[i11.at[r]], v11.at[r], sem))
        for cp_ in copies:
            cp_.wait()

        @pl.loop(0, _C // _L)
        def _interp(k):
            row = k // _VPR
            csl = pl.ds((k % _VPR) * _L, _L)
            a = v00[row, csl]
            b = v01[row, csl]
            c = v10[row, csl]
            d = v11[row, csl]
            sl = pl.ds(k * _L, _L)
            tv = t1b[sl]
            top = a + tv * (b - a)
            bot = c + tv * (d - c)
            ob[sl] = top + t0b[sl] * (bot - top)

        pltpu.sync_copy(ob, o_hbm.at[pl.ds(qbase, _C)])


def _interp_kernel(q_hbm, x_hbm, o_hbm, xb, i00, v4, t0b, t1b, ob, sem):
    base = _worker_id() * _QPW

    @pl.loop(0, _NCH)
    def _chunk(ch):
        qbase = base + ch * _C
        pltpu.sync_copy(x_hbm.at[pl.ds(2 * qbase, 2 * _C)], xb)

        @pl.loop(0, _C // _L)
        def _build(k):
            row = k // _VPR
            col = (k % _VPR) * _L
            xoff = 256 * (k // 8) + _L * (k % 8)
            x0 = xb[pl.ds(xoff, _L)]
            x1 = xb[pl.ds(xoff + 128, _L)]
            u = jnp.clip(x0, 0.0, 1.0) * float(_GRID - 1)
            v = jnp.clip(x1, 0.0, 1.0) * float(_GRID - 1)
            iv = jnp.minimum(u.astype(jnp.int32), _GRID - 2)
            jv = jnp.minimum(v.astype(jnp.int32), _GRID - 2)
            sl = pl.ds(k * _L, _L)
            t0b[sl] = u - iv.astype(jnp.float32)
            t1b[sl] = v - jv.astype(jnp.float32)
            p00 = ((iv >> 3) << 13) + ((jv >> 7) << 10) + ((iv & 7) << 7) \
                + (jv & 127)
            i00[row, pl.ds(col, _L)] = p00

        copies = [
            pltpu.async_copy(q_hbm.at[i00.at[r]], v4.at[r], sem)
            for r in range(_R)
        ]
        for cp in copies:
            cp.wait()

        @pl.loop(0, _C // _L)
        def _interp(k):
            r = k // _VPR
            qlanes = (k % _VPR) * _L + lax.iota(jnp.int32, _L)
            rsp = jnp.full((_L,), r, jnp.int32)
            a = plsc.load_gather(v4, [rsp, qlanes, jnp.full((_L,), 0, jnp.int32)])
            b = plsc.load_gather(v4, [rsp, qlanes, jnp.full((_L,), 1, jnp.int32)])
            c = plsc.load_gather(v4, [rsp, qlanes, jnp.full((_L,), 2, jnp.int32)])
            d = plsc.load_gather(v4, [rsp, qlanes, jnp.full((_L,), 3, jnp.int32)])
            sl = pl.ds(k * _L, _L)
            tv = t1b[sl]
            top = a + tv * (b - a)
            bot = c + tv * (d - c)
            ob[sl] = top + t0b[sl] * (bot - top)

        pltpu.sync_copy(ob, o_hbm.at[pl.ds(qbase, _C)])


def kernel(y, xs0, xs1, x):
    del xs0, xs1
    y_flat = y.reshape(128, 8, 8, 128).transpose(0, 2, 1, 3).reshape(-1)
    x_flat = x.reshape(16384, 128, 2).transpose(0, 2, 1).reshape(-1)
    mesh = plsc.VectorSubcoreMesh(core_axis_name="c", subcore_axis_name="s")
    cp = pltpu.CompilerParams()
    if "needs_layout_passes" in pltpu.CompilerParams.__dataclass_fields__:
        cp = dataclasses.replace(cp, needs_layout_passes=False)

    build = pl.kernel(
        _build_table_kernel,
        out_type=jax.ShapeDtypeStruct((4 * _NY,), jnp.float32),
        mesh=mesh,
        scratch_types=[
            pltpu.VMEM((_BIN,), jnp.float32),
            pltpu.VMEM((4 * _BCH,), jnp.float32),
            pltpu.SemaphoreType.DMA,
        ],
        compiler_params=cp,
    )
    q_table = build(y_flat)

    interp = pl.kernel(
        _interp_kernel_flat,
        out_type=jax.ShapeDtypeStruct((_NQ,), jnp.float32),
        mesh=mesh,
        scratch_types=[
            pltpu.VMEM((2 * _C,), jnp.float32),
            pltpu.VMEM((_R, _W), jnp.int32),
            pltpu.VMEM((_R, _W), jnp.int32),
            pltpu.VMEM((_R, _W), jnp.int32),
            pltpu.VMEM((_R, _W), jnp.int32),
            pltpu.VMEM((_R, _W), jnp.float32),
            pltpu.VMEM((_R, _W), jnp.float32),
            pltpu.VMEM((_R, _W), jnp.float32),
            pltpu.VMEM((_R, _W), jnp.float32),
            pltpu.VMEM((_C,), jnp.float32),
            pltpu.VMEM((_C,), jnp.float32),
            pltpu.VMEM((_C,), jnp.float32),
            pltpu.SemaphoreType.DMA,
        ],
        compiler_params=cp,
    )
    return interp(q_table, x_flat)


# speedup vs baseline: 2.2491x; 1.0338x over previous
import dataclasses

import jax
import jax.numpy as jnp
from jax import lax
from jax.experimental import pallas as pl
from jax.experimental.pallas import tpu as pltpu
from jax.experimental.pallas import tpu_sc as plsc

_GRID = 1024
_NY = _GRID * _GRID
_NQ = 2097152
_NCORES = 2
_NSUB = 16
_NW = _NCORES * _NSUB
_QPW = _NQ // _NW
_C = 2048
_NCH = _QPW // _C
_W = 128
_R = _C // _W
_L = 16
_VPR = _W // _L

_BCH = 4096
_BHALO = 8200
_BIN = _BCH + _BHALO
_BOFFMAX = _NY - _BIN
_SLAB = _NY // _NW


def _worker_id():
    return lax.axis_index("s") * _NCORES + lax.axis_index("c")


def _build_table_kernel(y_hbm, q_hbm, yb, qb, sem):
    del sem
    base = _worker_id() * _SLAB

    @pl.loop(0, _SLAB // _BCH)
    def _chunk(c):
        cb = base + c * _BCH
        off = jnp.minimum(cb, _BOFFMAX)
        sh = cb - off
        pltpu.sync_copy(y_hbm.at[pl.ds(off, _BIN)], yb)

        @pl.loop(0, _BCH // _L)
        def _pack(k):
            lanes = lax.iota(jnp.int32, _L)
            m = cb + k * _L + lanes
            lloc = sh + k * _L + lanes
            v0 = yb[pl.ds(sh + k * _L, _L)]
            dj = jnp.where((m & 127) == 127, 897, 1)
            di = jnp.where(((m >> 7) & 7) == 7, 7296, 128)
            s1 = jnp.minimum(lloc + dj, _BIN - 1)
            s2 = jnp.minimum(lloc + di, _BIN - 1)
            s3 = jnp.minimum(lloc + di + dj, _BIN - 1)
            v1 = plsc.load_gather(yb, [s1])
            v2 = plsc.load_gather(yb, [s2])
            v3 = plsc.load_gather(yb, [s3])
            q4 = 4 * (k * _L + lanes)
            plsc.store_scatter(qb, [q4], v0)
            plsc.store_scatter(qb, [q4 + 1], v1)
            plsc.store_scatter(qb, [q4 + 2], v2)
            plsc.store_scatter(qb, [q4 + 3], v3)

        pltpu.sync_copy(qb, q_hbm.at[pl.ds(4 * cb, 4 * _BCH)])


def _interp_kernel_pipe(q_hbm, x_hbm, o_hbm, xb, i00, i01, i10, i11,
                        v00, v01, v10, v11, t0b, t1b, ob, sem0, sem1):
    base = _worker_id() * _QPW
    sems = (sem0, sem1)

    def xin(ch, s):
        pltpu.sync_copy(x_hbm.at[pl.ds(2 * (base + ch * _C), 2 * _C)], xb.at[s])

    def build(ch, s):
        del ch

        @pl.loop(0, _C // _L)
        def _b(k):
            row = k // _VPR
            col = (k % _VPR) * _L
            xoff = 256 * (k // 8) + _L * (k % 8)
            x0 = xb[s, pl.ds(xoff, _L)]
            x1 = xb[s, pl.ds(xoff + 128, _L)]
            u = jnp.clip(x0, 0.0, 1.0) * float(_GRID - 1)
            v = jnp.clip(x1, 0.0, 1.0) * float(_GRID - 1)
            iv = jnp.minimum(u.astype(jnp.int32), _GRID - 2)
            jv = jnp.minimum(v.astype(jnp.int32), _GRID - 2)
            sl = pl.ds(k * _L, _L)
            t0b[s, sl] = u - iv.astype(jnp.float32)
            t1b[s, sl] = v - jv.astype(jnp.float32)
            p00 = ((iv >> 3) << 13) + ((jv >> 7) << 10) + ((iv & 7) << 7) \
                + (jv & 127)
            q4 = 4 * p00
            csl = pl.ds(col, _L)
            i00[s, row, csl] = q4
            i01[s, row, csl] = q4 + 1
            i10[s, row, csl] = q4 + 2
            i11[s, row, csl] = q4 + 3

    def fire(s):
        sem = sems[s]
        for r in range(_R):
            wsl = pl.ds(r * _W, _W)
            pltpu.async_copy(q_hbm.at[i00.at[s].at[r]], v00.at[s].at[wsl], sem)
            pltpu.async_copy(q_hbm.at[i01.at[s].at[r]], v01.at[s].at[wsl], sem)
            pltpu.async_copy(q_hbm.at[i10.at[s].at[r]], v10.at[s].at[wsl], sem)
            pltpu.async_copy(q_hbm.at[i11.at[s].at[r]], v11.at[s].at[wsl], sem)

    def drain(s):
        for vb in (v00, v01, v10, v11):
            pltpu.make_async_copy(q_hbm.at[pl.ds(0, _C)], vb.at[s], sems[s]).wait()

    def interp(s):
        @pl.loop(0, _C // _L)
        def _i(k):
            sl = pl.ds(k * _L, _L)
            a = v00[s, sl]
            b = v01[s, sl]
            c = v10[s, sl]
            d = v11[s, sl]
            tv = t1b[s, sl]
            top = a + tv * (b - a)
            bot = c + tv * (d - c)
            ob[s, sl] = top + t0b[s, sl] * (bot - top)

    def out(ch, s):
        pltpu.sync_copy(ob.at[s], o_hbm.at[pl.ds(base + ch * _C, _C)])

    xin(0, 0)
    build(0, 0)
    fire(0)

    @pl.loop(0, (_NCH - 2) // 2)
    def _steady(g):
        ch = 2 * g + 1
        xin(ch, 1)
        build(ch, 1)
        drain(0)
        fire(1)
        interp(0)
        out(ch - 1, 0)
        xin(ch + 1, 0)
        build(ch + 1, 0)
        drain(1)
        fire(0)
        interp(1)
        out(ch, 1)

    ch_last = _NCH - 1
    xin(ch_last, 1)
    build(ch_last, 1)
    drain(0)
    fire(1)
    interp(0)
    out(ch_last - 1, 0)
    drain(1)
    interp(1)
    out(ch_last, 1)


def kernel(y, xs0, xs1, x):
    del xs0, xs1
    y_flat = y.reshape(128, 8, 8, 128).transpose(0, 2, 1, 3).reshape(-1)
    x_flat = x.reshape(16384, 128, 2).transpose(0, 2, 1).reshape(-1)
    mesh = plsc.VectorSubcoreMesh(core_axis_name="c", subcore_axis_name="s")
    cp = pltpu.CompilerParams()
    if "needs_layout_passes" in pltpu.CompilerParams.__dataclass_fields__:
        cp = dataclasses.replace(cp, needs_layout_passes=False)

    build = pl.kernel(
        _build_table_kernel,
        out_type=jax.ShapeDtypeStruct((4 * _NY,), jnp.float32),
        mesh=mesh,
        scratch_types=[
            pltpu.VMEM((_BIN,), jnp.float32),
            pltpu.VMEM((4 * _BCH,), jnp.float32),
            pltpu.SemaphoreType.DMA,
        ],
        compiler_params=cp,
    )
    q_table = build(y_flat)

    interp = pl.kernel(
        _interp_kernel_pipe,
        out_type=jax.ShapeDtypeStruct((_NQ,), jnp.float32),
        mesh=mesh,
        scratch_types=[
            pltpu.VMEM((2, 2 * _C), jnp.float32),
            pltpu.VMEM((2, _R, _W), jnp.int32),
            pltpu.VMEM((2, _R, _W), jnp.int32),
            pltpu.VMEM((2, _R, _W), jnp.int32),
            pltpu.VMEM((2, _R, _W), jnp.int32),
            pltpu.VMEM((2, _C), jnp.float32),
            pltpu.VMEM((2, _C), jnp.float32),
            pltpu.VMEM((2, _C), jnp.float32),
            pltpu.VMEM((2, _C), jnp.float32),
            pltpu.VMEM((2, _C), jnp.float32),
            pltpu.VMEM((2, _C), jnp.float32),
            pltpu.VMEM((2, _C), jnp.float32),
            pltpu.SemaphoreType.DMA,
            pltpu.SemaphoreType.DMA,
        ],
        compiler_params=cp,
    )
    return interp(q_table, x_flat)


# speedup vs baseline: 2.2867x; 1.0167x over previous
import dataclasses

import jax
import jax.numpy as jnp
from jax import lax
from jax.experimental import pallas as pl
from jax.experimental.pallas import tpu as pltpu
from jax.experimental.pallas import tpu_sc as plsc

_GRID = 1024
_NY = _GRID * _GRID
_NQ = 2097152
_NCORES = 2
_NSUB = 16
_NW = _NCORES * _NSUB
_QPW = _NQ // _NW
_C = 2048
_NCH = _QPW // _C
_W = 128
_R = _C // _W
_L = 16
_VPR = _W // _L

_BCH = 4096
_BHALO = 8200
_BIN = _BCH + _BHALO
_BOFFMAX = _NY - _BIN
_SLAB = _NY // _NW


def _worker_id():
    return lax.axis_index("s") * _NCORES + lax.axis_index("c")


def _build_table_kernel(y_hbm, q_hbm, yb, qb, sem):
    del sem
    base = _worker_id() * _SLAB

    @pl.loop(0, _SLAB // _BCH)
    def _chunk(c):
        cb = base + c * _BCH
        off = jnp.minimum(cb, _BOFFMAX)
        sh = cb - off
        pltpu.sync_copy(y_hbm.at[pl.ds(off, _BIN)], yb)

        @pl.loop(0, _BCH // _L)
        def _pack(k):
            lanes = lax.iota(jnp.int32, _L)
            m = cb + k * _L + lanes
            lloc = sh + k * _L + lanes
            v0 = yb[pl.ds(sh + k * _L, _L)]
            dj = jnp.where((m & 127) == 127, 897, 1)
            di = jnp.where(((m >> 7) & 7) == 7, 7296, 128)
            s1 = jnp.minimum(lloc + dj, _BIN - 1)
            s2 = jnp.minimum(lloc + di, _BIN - 1)
            s3 = jnp.minimum(lloc + di + dj, _BIN - 1)
            v1 = plsc.load_gather(yb, [s1])
            v2 = plsc.load_gather(yb, [s2])
            v3 = plsc.load_gather(yb, [s3])
            q4 = 4 * (k * _L + lanes)
            plsc.store_scatter(qb, [q4], v0)
            plsc.store_scatter(qb, [q4 + 1], v1)
            plsc.store_scatter(qb, [q4 + 2], v2)
            plsc.store_scatter(qb, [q4 + 3], v3)

        pltpu.sync_copy(qb, q_hbm.at[pl.ds(4 * cb, 4 * _BCH)])


def _interp_kernel_quad(q_hbm, x_hbm, o_hbm, xb, iq0, iq1, vq0, vq1,
                        t0b, t1b, ob, sem0, sem1):
    base = _worker_id() * _QPW
    sems = (sem0, sem1)
    iqs = (iq0, iq1)
    vqs = (vq0, vq1)
    _QR = 4 * _C // _W

    def xin(ch, s):
        pltpu.sync_copy(x_hbm.at[pl.ds(2 * (base + ch * _C), 2 * _C)], xb.at[s])

    def build(ch, s):
        del ch
        iq = iqs[s]

        @pl.loop(0, _C // _L)
        def _b(k):
            xoff = 256 * (k // 8) + _L * (k % 8)
            x0 = xb[s, pl.ds(xoff, _L)]
            x1 = xb[s, pl.ds(xoff + 128, _L)]
            u = jnp.clip(x0, 0.0, 1.0) * float(_GRID - 1)
            v = jnp.clip(x1, 0.0, 1.0) * float(_GRID - 1)
            iv = jnp.minimum(u.astype(jnp.int32), _GRID - 2)
            jv = jnp.minimum(v.astype(jnp.int32), _GRID - 2)
            sl = pl.ds(k * _L, _L)
            t0b[s, sl] = u - iv.astype(jnp.float32)
            t1b[s, sl] = v - jv.astype(jnp.float32)
            p00 = ((iv >> 3) << 13) + ((jv >> 7) << 10) + ((iv & 7) << 7) \
                + (jv & 127)
            q4 = 4 * p00
            lanes = lax.iota(jnp.int32, _L)
            row = jnp.full((_L,), k // 2, jnp.int32)
            colbase = 64 * (k % 2) + 4 * lanes
            plsc.store_scatter(iq, [row, colbase], q4)
            plsc.store_scatter(iq, [row, colbase + 1], q4 + 1)
            plsc.store_scatter(iq, [row, colbase + 2], q4 + 2)
            plsc.store_scatter(iq, [row, colbase + 3], q4 + 3)

    def fire(s):
        sem = sems[s]
        for r in range(_QR):
            pltpu.async_copy(q_hbm.at[iqs[s].at[r]],
                             vqs[s].at[pl.ds(r * _W, _W)], sem)

    def drain(s):
        pltpu.make_async_copy(q_hbm.at[pl.ds(0, 4 * _C)], vqs[s], sems[s]).wait()

    def interp(s):
        vq = vqs[s]

        @pl.loop(0, _C // _L)
        def _i(k):
            qi = 64 * k + 4 * lax.iota(jnp.int32, _L)
            a = plsc.load_gather(vq, [qi])
            b = plsc.load_gather(vq, [qi + 1])
            c = plsc.load_gather(vq, [qi + 2])
            d = plsc.load_gather(vq, [qi + 3])
            sl = pl.ds(k * _L, _L)
            tv = t1b[s, sl]
            top = a + tv * (b - a)
            bot = c + tv * (d - c)
            ob[s, sl] = top + t0b[s, sl] * (bot - top)

    def out(ch, s):
        pltpu.sync_copy(ob.at[s], o_hbm.at[pl.ds(base + ch * _C, _C)])

    xin(0, 0)
    build(0, 0)
    fire(0)

    @pl.loop(0, (_NCH - 2) // 2)
    def _steady(g):
        ch = 2 * g + 1
        xin(ch, 1)
        build(ch, 1)
        drain(0)
        fire(1)
        interp(0)
        out(ch - 1, 0)
        xin(ch + 1, 0)
        build(ch + 1, 0)
        drain(1)
        fire(0)
        interp(1)
        out(ch, 1)

    ch_last = _NCH - 1
    xin(ch_last, 1)
    build(ch_last, 1)
    drain(0)
    fire(1)
    interp(0)
    out(ch_last - 1, 0)
    drain(1)
    interp(1)
    out(ch_last, 1)


def _interp_kernel_pipe(q_hbm, x_hbm, o_hbm, xb, i00, i01, i10, i11,
                        v00, v01, v10, v11, t0b, t1b, ob, sem0, sem1):
    base = _worker_id() * _QPW
    sems = (sem0, sem1)

    def xin(ch, s):
        pltpu.sync_copy(x_hbm.at[pl.ds(2 * (base + ch * _C), 2 * _C)], xb.at[s])

    def build(ch, s):
        del ch

        @pl.loop(0, _C // _L)
        def _b(k):
            row = k // _VPR
            col = (k % _VPR) * _L
            xoff = 256 * (k // 8) + _L * (k % 8)
            x0 = xb[s, pl.ds(xoff, _L)]
            x1 = xb[s, pl.ds(xoff + 128, _L)]
            u = jnp.clip(x0, 0.0, 1.0) * float(_GRID - 1)
            v = jnp.clip(x1, 0.0, 1.0) * float(_GRID - 1)
            iv = jnp.minimum(u.astype(jnp.int32), _GRID - 2)
            jv = jnp.minimum(v.astype(jnp.int32), _GRID - 2)
            sl = pl.ds(k * _L, _L)
            t0b[s, sl] = u - iv.astype(jnp.float32)
            t1b[s, sl] = v - jv.astype(jnp.float32)
            p00 = ((iv >> 3) << 13) + ((jv >> 7) << 10) + ((iv & 7) << 7) \
                + (jv & 127)
            q4 = 4 * p00
            csl = pl.ds(col, _L)
            i00[s, row, csl] = q4
            i01[s, row, csl] = q4 + 1
            i10[s, row, csl] = q4 + 2
            i11[s, row, csl] = q4 + 3

    def fire(s):
        sem = sems[s]
        for r in range(_R):
            wsl = pl.ds(r * _W, _W)
            pltpu.async_copy(q_hbm.at[i00.at[s].at[r]], v00.at[s].at[wsl], sem)
            pltpu.async_copy(q_hbm.at[i01.at[s].at[r]], v01.at[s].at[wsl], sem)
            pltpu.async_copy(q_hbm.at[i10.at[s].at[r]], v10.at[s].at[wsl], sem)
            pltpu.async_copy(q_hbm.at[i11.at[s].at[r]], v11.at[s].at[wsl], sem)

    def drain(s):
        for vb in (v00, v01, v10, v11):
            pltpu.make_async_copy(q_hbm.at[pl.ds(0, _C)], vb.at[s], sems[s]).wait()

    def interp(s):
        @pl.loop(0, _C // _L)
        def _i(k):
            sl = pl.ds(k * _L, _L)
            a = v00[s, sl]
            b = v01[s, sl]
            c = v10[s, sl]
            d = v11[s, sl]
            tv = t1b[s, sl]
            top = a + tv * (b - a)
            bot = c + tv * (d - c)
            ob[s, sl] = top + t0b[s, sl] * (bot - top)

    def out(ch, s):
        pltpu.sync_copy(ob.at[s], o_hbm.at[pl.ds(base + ch * _C, _C)])

    xin(0, 0)
    build(0, 0)
    fire(0)

    @pl.loop(0, (_NCH - 2) // 2)
    def _steady(g):
        ch = 2 * g + 1
        xin(ch, 1)
        build(ch, 1)
        drain(0)
        fire(1)
        interp(0)
        out(ch - 1, 0)
        xin(ch + 1, 0)
        build(ch + 1, 0)
        drain(1)
        fire(0)
        interp(1)
        out(ch, 1)

    ch_last = _NCH - 1
    xin(ch_last, 1)
    build(ch_last, 1)
    drain(0)
    fire(1)
    interp(0)
    out(ch_last - 1, 0)
    drain(1)
    interp(1)
    out(ch_last, 1)


def kernel(y, xs0, xs1, x):
    del xs0, xs1
    y_flat = y.reshape(128, 8, 8, 128).transpose(0, 2, 1, 3).reshape(-1)
    x_flat = x.reshape(16384, 128, 2).transpose(0, 2, 1).reshape(-1)
    mesh = plsc.VectorSubcoreMesh(core_axis_name="c", subcore_axis_name="s")
    cp = pltpu.CompilerParams()
    if "needs_layout_passes" in pltpu.CompilerParams.__dataclass_fields__:
        cp = dataclasses.replace(cp, needs_layout_passes=False)

    build = pl.kernel(
        _build_table_kernel,
        out_type=jax.ShapeDtypeStruct((4 * _NY,), jnp.float32),
        mesh=mesh,
        scratch_types=[
            pltpu.VMEM((_BIN,), jnp.float32),
            pltpu.VMEM((4 * _BCH,), jnp.float32),
            pltpu.SemaphoreType.DMA,
        ],
        compiler_params=cp,
    )
    q_table = build(y_flat)

    interp = pl.kernel(
        _interp_kernel_quad,
        out_type=jax.ShapeDtypeStruct((_NQ,), jnp.float32),
        mesh=mesh,
        scratch_types=[
            pltpu.VMEM((2, 2 * _C), jnp.float32),
            pltpu.VMEM((4 * _C // _W, _W), jnp.int32),
            pltpu.VMEM((4 * _C // _W, _W), jnp.int32),
            pltpu.VMEM((4 * _C,), jnp.float32),
            pltpu.VMEM((4 * _C,), jnp.float32),
            pltpu.VMEM((2, _C), jnp.float32),
            pltpu.VMEM((2, _C), jnp.float32),
            pltpu.VMEM((2, _C), jnp.float32),
            pltpu.SemaphoreType.DMA,
            pltpu.SemaphoreType.DMA,
        ],
        compiler_params=cp,
    )
    return interp(q_table, x_flat)
